# baseline (speedup 1.0000x reference)
import functools

import jax
import jax.numpy as jnp
from jax import lax
from jax.experimental import pallas as pl
from jax.experimental.pallas import tpu as pltpu
from jax.experimental.pallas import tpu_sc as plsc

_INFO = plsc.get_sparse_core_info()
_NC = _INFO.num_cores
_NS = _INFO.num_subcores
_NW = _NC * _NS
_L = _INFO.num_lanes


@functools.partial(jax.jit, static_argnames=("bw",))
def _sc_gather_t(xt, weight, *, bw):
    h, b = xt.shape
    v, d = weight.shape
    dblk = d // _L
    bblk = bw // _L
    mesh = plsc.VectorSubcoreMesh(core_axis_name="c", subcore_axis_name="s")

    @functools.partial(
        pl.kernel,
        mesh=mesh,
        out_type=jax.ShapeDtypeStruct((h, d, b), jnp.float32),
        scratch_types=[
            pltpu.VMEM((h, bw), jnp.int32),
            pltpu.VMEM((bw, d), jnp.float32),
            pltpu.VMEM((bw, d), jnp.float32),
            pltpu.VMEM((d, bw), jnp.float32),
            pltpu.VMEM((d, bw), jnp.float32),
            pltpu.SemaphoreType.DMA,
            pltpu.SemaphoreType.DMA,
        ],
        compiler_params=pltpu.CompilerParams(
            use_tc_tiling_on_sc=False,
            needs_layout_passes=False,
            disable_bounds_checks=True,
        ),
    )
    def k(xt_hbm, table_hbm, out_hbm, idx_v, rows0, rows1, rt0, rt1, s0, s1):
        rows_b = (rows0, rows1)
        rt_b = (rt0, rt1)
        sems = (s0, s1)
        wid = lax.axis_index("s") * _NC + lax.axis_index("c")
        b0 = wid * bw

        pltpu.sync_copy(xt_hbm.at[:, pl.ds(b0, bw)], idx_v)

        def start(hh, s):
            pltpu.async_copy(
                table_hbm.at[idx_v.at[hh]], rows_b[s], sems[s]
            )

        def wait(s):
            pltpu.make_async_copy(
                table_hbm.at[idx_v.at[0]], rows_b[s], sems[s]
            ).wait()

        lane = lax.iota(jnp.int32, _L)
        d_rows = [kk * _L + lane for kk in range(dblk)]

        def transpose(s):
            rows = rows_b[s]
            rt = rt_b[s]

            def tbody(bb):
                colv = jnp.broadcast_to(bb, (_L,))
                for kk in range(dblk):
                    val = rows[bb, pl.ds(kk * _L, _L)]
                    plsc.store_scatter(rt, [d_rows[kk], colv], val)

            plsc.parallel_loop(0, bw, 1, unroll=8)(tbody)

        start(0, 0)
        start(1, 1)

        def body(i, _):
            h0 = i * 2
            for s in range(2):
                hh = h0 + s
                wait(s)
                transpose(s)

                @pl.when(hh + 2 < h)
                def _prefetch():
                    start(hh + 2, s)

                pltpu.sync_copy(
                    rt_b[s], out_hbm.at[hh, :, pl.ds(b0, bw)]
                )
            return _

        lax.fori_loop(0, h // 2, body, 0)

    return k(xt, weight)


def kernel(x, weight):
    b, h = x.shape
    v, d = weight.shape
    xt = x.T.astype(jnp.int32)
    out_t = _sc_gather_t(xt, weight, bw=b // _NW)
    return out_t.transpose(2, 0, 1)


# speedup vs baseline: 2.3912x; 2.3912x over previous
import functools

import jax
import jax.numpy as jnp
from jax import lax
from jax.experimental import pallas as pl
from jax.experimental.pallas import tpu as pltpu
from jax.experimental.pallas import tpu_sc as plsc

_INFO = plsc.get_sparse_core_info()
_NC = _INFO.num_cores
_NS = _INFO.num_subcores
_NW = _NC * _NS
_L = _INFO.num_lanes


@functools.partial(jax.jit, static_argnames=("bw", "d"))
def _sc_gather_t(xt, wp, *, bw, d):
    h, b = xt.shape
    v, dp = wp.shape
    dblk = d // _L
    mesh = plsc.VectorSubcoreMesh(core_axis_name="c", subcore_axis_name="s")

    @functools.partial(
        pl.kernel,
        mesh=mesh,
        out_type=jax.ShapeDtypeStruct((h, d, b), jnp.float32),
        scratch_types=[
            pltpu.VMEM((h, bw), jnp.int32),
            pltpu.VMEM((bw, dp), jnp.float32),
            pltpu.VMEM((bw, dp), jnp.float32),
            pltpu.VMEM(((bw // _L) * dblk * 272,), jnp.float32),
            pltpu.VMEM((d, bw), jnp.float32),
            pltpu.VMEM((d, bw), jnp.float32),
            pltpu.SemaphoreType.DMA,
            pltpu.SemaphoreType.DMA,
            pltpu.SemaphoreType.DMA,
            pltpu.SemaphoreType.DMA,
        ],
        compiler_params=pltpu.CompilerParams(
            use_tc_tiling_on_sc=True,
            needs_layout_passes=False,
            disable_bounds_checks=True,
        ),
    )
    def k(
        xt_hbm, table_hbm, out_hbm,
        idx_v, rows0, rows1, rtb, rt0, rt1, s0, s1, w0, w1,
    ):
        rows_b = (rows0, rows1)
        rt_b = (rt0, rt1)
        sems = (s0, s1)
        wsem = (w0, w1)
        wid = lax.axis_index("s") * _NC + lax.axis_index("c")
        b0 = wid * bw

        pltpu.sync_copy(xt_hbm.at[:, pl.ds(b0, bw)], idx_v)

        def start(hh, s):
            pltpu.async_copy(
                table_hbm.at[idx_v.at[hh]], rows_b[s], sems[s]
            )

        def wait(s):
            pltpu.make_async_copy(
                table_hbm.at[idx_v.at[0]], rows_b[s], sems[s]
            ).wait()

        def wait_w(s):
            pltpu.make_async_copy(
                rt_b[s], out_hbm.at[0, :, pl.ds(b0, bw)], wsem[s]
            ).wait()

        lane = lax.iota(jnp.int32, _L)
        lane17 = lane * 17

        def transpose(s):
            rows = rows_b[s]
            rt = rt_b[s]

            def stage1(bb):
                kb = bb // _L
                rr = lax.rem(bb, _L)
                base = kb * (dblk * 272) + rr * 17
                for kk in range(dblk):
                    val = rows[bb, pl.ds(kk * _L, _L)]
                    rtb[pl.ds(base + kk * 272, _L)] = val

            plsc.parallel_loop(0, bw, 1, unroll=8)(stage1)

            def stage2(j):
                kb = j // _L
                q = lax.rem(j, _L)
                for kk in range(dblk):
                    base = kb * (dblk * 272) + kk * 272 + q
                    val = plsc.load_gather(rtb, [lane17 + base])
                    rt[kk * _L + q, pl.ds(kb * _L, _L)] = val

            plsc.parallel_loop(0, bw, 1, unroll=8)(stage2)

        start(0, 0)
        start(1, 1)

        def body(i, _):
            h0 = i * 2
            for s in range(2):
                hh = h0 + s
                wait(s)

                @pl.when(hh >= 2)
                def _drain_prev():
                    wait_w(s)

                transpose(s)

                @pl.when(hh + 2 < h)
                def _prefetch():
                    start(hh + 2, s)

                pltpu.async_copy(
                    rt_b[s], out_hbm.at[hh, :, pl.ds(b0, bw)], wsem[s]
                )
            return _

        lax.fori_loop(0, h // 2, body, 0)
        wait_w(0)
        wait_w(1)

    return k(xt, wp)


def kernel(x, weight):
    b, h = x.shape
    v, d = weight.shape
    xt = x.T.astype(jnp.int32)
    dp = 128 * ((d + 127) // 128)
    wp = jnp.pad(weight, ((0, 0), (0, dp - d)))
    out_t = _sc_gather_t(xt, wp, bw=b // _NW, d=d)
    return out_t.transpose(2, 0, 1)


# speedup vs baseline: 2.4049x; 1.0057x over previous
import functools

import jax
import jax.numpy as jnp
from jax import lax
from jax.experimental import pallas as pl
from jax.experimental.pallas import tpu as pltpu
from jax.experimental.pallas import tpu_sc as plsc

_INFO = plsc.get_sparse_core_info()
_NC = _INFO.num_cores
_NS = _INFO.num_subcores
_NW = _NC * _NS
_L = _INFO.num_lanes


@functools.partial(jax.jit, static_argnames=("bw", "d"))
def _sc_gather_t(xt, wp, *, bw, d):
    h, b = xt.shape
    v, dp = wp.shape
    dblk = d // _L
    mesh = plsc.VectorSubcoreMesh(core_axis_name="c", subcore_axis_name="s")

    @functools.partial(
        pl.kernel,
        mesh=mesh,
        out_type=jax.ShapeDtypeStruct((h, d, b), jnp.float32),
        scratch_types=[
            pltpu.VMEM((h, bw), jnp.int32),
            pltpu.VMEM((bw, dp), jnp.float32),
            pltpu.VMEM((bw, dp), jnp.float32),
            pltpu.VMEM(((bw // _L) * dblk * 272,), jnp.float32),
            pltpu.VMEM((d, bw), jnp.float32),
            pltpu.VMEM((d, bw), jnp.float32),
            pltpu.SemaphoreType.DMA,
            pltpu.SemaphoreType.DMA,
        ],
        compiler_params=pltpu.CompilerParams(
            use_tc_tiling_on_sc=True,
            needs_layout_passes=False,
            disable_bounds_checks=True,
        ),
    )
    def k(
        xt_hbm, table_hbm, out_hbm,
        idx_v, rows0, rows1, rtb, rt0, rt1, s0, s1,
    ):
        rows_b = (rows0, rows1)
        rt_b = (rt0, rt1)
        sems = (s0, s1)
        wid = lax.axis_index("s") * _NC + lax.axis_index("c")
        b0 = wid * bw

        pltpu.sync_copy(xt_hbm.at[:, pl.ds(b0, bw)], idx_v)

        def start(hh, s):
            pltpu.async_copy(
                table_hbm.at[idx_v.at[hh]], rows_b[s], sems[s]
            )

        def wait(s):
            pltpu.make_async_copy(
                table_hbm.at[idx_v.at[0]], rows_b[s], sems[s]
            ).wait()

        lane = lax.iota(jnp.int32, _L)
        lane17 = lane * 17

        def transpose(s):
            rows = rows_b[s]
            rt = rt_b[s]

            def stage1(bb):
                kb = bb // _L
                rr = lax.rem(bb, _L)
                base = kb * (dblk * 272) + rr * 17
                for kk in range(dblk):
                    val = rows[bb, pl.ds(kk * _L, _L)]
                    rtb[pl.ds(base + kk * 272, _L)] = val

            plsc.parallel_loop(0, bw, 1, unroll=8)(stage1)

            def stage2(j):
                kb = j // _L
                q = lax.rem(j, _L)
                for kk in range(dblk):
                    base = kb * (dblk * 272) + kk * 272 + q
                    val = plsc.load_gather(rtb, [lane17 + base])
                    rt[kk * _L + q, pl.ds(kb * _L, _L)] = val

            plsc.parallel_loop(0, bw, 1, unroll=8)(stage2)

        start(0, 0)
        start(1, 1)

        def body(i, _):
            h0 = i * 2
            for s in range(2):
                hh = h0 + s
                wait(s)
                transpose(s)

                @pl.when(hh + 2 < h)
                def _prefetch():
                    start(hh + 2, s)

                pltpu.sync_copy(
                    rt_b[s], out_hbm.at[hh, :, pl.ds(b0, bw)]
                )
            return _

        lax.fori_loop(0, h // 2, body, 0)

    return k(xt, wp)


def kernel(x, weight):
    b, h = x.shape
    v, d = weight.shape
    xt = x.T.astype(jnp.int32)
    dp = 128 * ((d + 127) // 128)
    wp = jnp.pad(weight, ((0, 0), (0, dp - d)))
    out_t = _sc_gather_t(xt, wp, bw=b // _NW, d=d)
    return out_t.transpose(2, 0, 1)
